# baseline (speedup 1.0000x reference)
import functools

import jax
import jax.numpy as jnp
from jax import lax
from jax.experimental import pallas as pl
from jax.experimental.pallas import tpu as pltpu
from jax.experimental.pallas import tpu_sc as plsc

D = 64
N = 100000
B = 16384
BH = B // 2
NC = 2
NS = 16
NW = NC * NS
PER_W = B // NW
CH = 128
NCH = PER_W // CH

_MESH = plsc.VectorSubcoreMesh(
    core_axis_name="c", subcore_axis_name="s", num_cores=NC, num_subcores=NS
)

_G_ET = (
    (0, ((0, 0),)),
    (1, ((1, 0),)),
    (2, ((2, 0),)),
    (3, ((3, 0),)),
    (4, ((4, 0),)),
    (5, ((5, 1),)),
    (6, ((6, 1),)),
)
_G_BC = (
    (0, ((0, 0), (1, 1))),
    (1, ((2, 0), (3, 1))),
    (2, ((4, 0), (5, 1))),
)
_G_HR = (
    (0, ((0, 0),)),
    (1, ((1, 0),)),
    (2, ((2, 1),)),
    (3, ((3, 1),)),
)


_EC = 2048
_NEB = (N + _EC - 1) // _EC


_TC_MESH = pltpu.create_tensorcore_mesh("tc", num_cores=2)


@functools.partial(
    pl.kernel,
    out_type=jax.ShapeDtypeStruct((N, 2 * D), jnp.float32),
    mesh=_TC_MESH,
)
def _repack(aT, bT, p_hbm):
    def body(a_vmem, b_vmem, o_vmem):
        o_vmem[:, :D] = a_vmem[...].T
        o_vmem[:, D:] = b_vmem[...].T

    pltpu.emit_pipeline(
        body,
        grid=(_NEB,),
        in_specs=[pl.BlockSpec((D, _EC), lambda i: (0, i))] * 2,
        out_specs=[pl.BlockSpec((_EC, 2 * D), lambda i: (i, 0))],
        core_axis_name="tc",
        dimension_semantics=(pltpu.PARALLEL,),
    )(aT, bT, p_hbm)


def _make_sc_gather(gathers, n_idx, n_out):
    @functools.partial(
        pl.kernel,
        out_type=jax.ShapeDtypeStruct((n_out, BH, 2 * D), jnp.float32),
        mesh=_MESH,
        scratch_types=[
            pltpu.VMEM((n_idx, PER_W), jnp.int32),
            pltpu.VMEM((2, CH, 2 * D), jnp.float32),
            pltpu.SemaphoreType.DMA,
            pltpu.SemaphoreType.DMA,
            pltpu.SemaphoreType.DMA,
        ],
        compiler_params=pltpu.CompilerParams(use_tc_tiling_on_sc=False),
    )
    def sc_gather(tbl, idx_hbm, out, idx_v, rows, sem_i, sem_a, sem_b):
        wid = lax.axis_index("s") * NC + lax.axis_index("c")
        base = wid * PER_W
        half = base // BH
        row0 = base - half * BH
        pltpu.async_copy(idx_hbm.at[:, pl.ds(base, PER_W)], idx_v, sem_i
                         ).wait()
        for ii, outs in gathers:

            @pl.loop(0, NCH, step=2)
            def _(c, ii=ii, outs=outs):
                o0 = c * CH
                o1 = o0 + CH
                cp_a = pltpu.async_copy(
                    tbl.at[idx_v.at[ii, pl.ds(o0, CH)]], rows.at[0], sem_a)
                cp_b = pltpu.async_copy(
                    tbl.at[idx_v.at[ii, pl.ds(o1, CH)]], rows.at[1], sem_b)
                cp_a.wait()
                for g, h in outs:
                    pltpu.sync_copy(
                        rows.at[0, :, pl.ds(h * D, D)],
                        out.at[g, pl.ds(row0 + o0, CH),
                               pl.ds(half * D, D)])
                cp_b.wait()
                for g, h in outs:
                    pltpu.sync_copy(
                        rows.at[1, :, pl.ds(h * D, D)],
                        out.at[g, pl.ds(row0 + o1, CH),
                               pl.ds(half * D, D)])

    return sc_gather


_sc_et = _make_sc_gather(_G_ET, 7, 7)
_sc_bc = _make_sc_gather(_G_BC, 3, 6)
_sc_hr = _make_sc_gather(_G_HR, 4, 4)


_RH = 512


def _tc_body(m_ref, et_ref, bc_ref, hr_ref, o1, o2, o3, o4, o5, o6):
    m = m_ref[0, 0]
    one = jnp.float32(1.0)
    et = et_ref[...]
    bc = bc_ref[...]
    hr = hr_ref[...]

    for h in (0, 1):
        sl = slice(h * D, h * D + D)
        aBHEE, aBTEE = et[0][:, sl], et[1][:, sl]
        nABHEE, nABTEE = et[2][:, sl], et[3][:, sl]
        uniqEE = et[4][:, sl]
        aTailE, nTailE = et[5][:, sl], et[6][:, sl]
        aBCHE, aBCTE = bc[0][:, sl], bc[1][:, sl]
        nABCHE, nABCTE = bc[2][:, sl], bc[3][:, sl]
        uniqBCHE, uniqBCTE = bc[4][:, sl], bc[5][:, sl]
        aHeadE, nHeadE = hr[0][:, sl], hr[1][:, sl]
        aRelE, nRelE = hr[2][:, sl], hr[3][:, sl]

        rs = lambda v: jnp.sum(v, axis=1)
        tmpBE2CH = (one - aBCHE) * aBHEE
        tmpBE2CT = (one - aBCTE) * aBTEE
        tmpTransE = rs(jnp.abs(aHeadE + aRelE - aTailE))
        o1[h, :] = (rs(tmpBE2CH * tmpBE2CH) + rs(tmpBE2CT * tmpBE2CT)
                    + tmpTransE)
        tmpNBE2CH = (one - nABCHE) * nABHEE
        tmpNBE2CT = (one - nABCTE) * nABTEE
        tmpNTransE = rs(jnp.abs(nHeadE + nRelE - nTailE))
        tmpNBL = (rs(tmpNBE2CH * tmpNBE2CH) + rs(tmpNBE2CT * tmpNBE2CT)
                  + tmpNTransE)
        o2[h, :] = jnp.maximum(m - tmpNBL, 0.0)
        tmpE = rs(uniqEE * uniqEE) - one
        o3[h, :] = tmpE * tmpE
        tmpBCH = uniqBCHE * (one - uniqBCHE)
        tmpBCT = uniqBCTE * (one - uniqBCTE)
        o4[h, :] = rs(tmpBCH * tmpBCH) + rs(tmpBCT * tmpBCT)
        o5[h, :] = (jnp.maximum(one - rs(jnp.abs(uniqBCHE)), 0.0)
                    + jnp.maximum(one - rs(jnp.abs(uniqBCTE)), 0.0))
        o6[h, :] = jnp.maximum(m + tmpTransE - tmpNTransE, 0.0)


def _tc_compute(margin2d, g_et, g_bc, g_hr):
    halves = jax.ShapeDtypeStruct((2, BH), jnp.float32)
    blk = lambda n: pl.BlockSpec((n, _RH, 2 * D), lambda i: (0, i, 0))
    return pl.pallas_call(
        _tc_body,
        grid=(BH // _RH,),
        in_specs=[
            pl.BlockSpec((1, 1), lambda i: (0, 0)),
            blk(7), blk(6), blk(4),
        ],
        out_specs=[pl.BlockSpec((2, _RH), lambda i: (0, i))] * 6,
        out_shape=[halves] * 6,
        compiler_params=pltpu.CompilerParams(
            dimension_semantics=("arbitrary",),
        ),
    )(margin2d, g_et, g_bc, g_hr)


def kernel(aBHE, aBTE, aBC, aHead, aTail, aRelation, nABHE, nABTE, nABC,
           nHead, nTail, nRelation, uniqE, uniqBC, lossMargin, device,
           entityEmbed, bConceptHEmbed, bConceptTEmbed, headEmbed,
           tailEmbed, relationEmbed):
    i32 = lambda a: a.astype(jnp.int32)
    idx_et = jnp.stack([i32(aBHE), i32(aBTE), i32(nABHE), i32(nABTE),
                        i32(uniqE), i32(aTail), i32(nTail)])
    idx_bc = jnp.stack([i32(aBC), i32(nABC), i32(uniqBC)])
    idx_hr = jnp.stack([i32(aHead), i32(nHead), i32(aRelation),
                        i32(nRelation)])
    sw = lambda t: jnp.swapaxes(t, 0, 1)
    p_et = _repack(sw(entityEmbed), sw(tailEmbed))
    g_et = _sc_et(p_et, idx_et)
    p_bc = _repack(sw(bConceptHEmbed), sw(bConceptTEmbed))
    g_bc = _sc_bc(p_bc, idx_bc)
    p_hr = _repack(sw(headEmbed), sw(relationEmbed))
    g_hr = _sc_hr(p_hr, idx_hr)
    margin2d = jnp.asarray(lossMargin, jnp.float32).reshape(1, 1)
    o = _tc_compute(margin2d, g_et, g_bc, g_hr)
    join = lambda t: jnp.concatenate([t[0], t[1]])
    return tuple(join(t) for t in o)


# speedup vs baseline: 1.1822x; 1.1822x over previous
import functools

import jax
import jax.numpy as jnp
from jax import lax
from jax.experimental import pallas as pl
from jax.experimental.pallas import tpu as pltpu
from jax.experimental.pallas import tpu_sc as plsc

D = 64
N = 100000
B = 16384
BH = B // 2
NC = 2
NS = 16
NW = NC * NS
PER_W = B // NW
CH = 128
NCH = PER_W // CH

_MESH = plsc.VectorSubcoreMesh(
    core_axis_name="c", subcore_axis_name="s", num_cores=NC, num_subcores=NS
)

_G_ET = (
    (0, ((0, 0),)),
    (1, ((1, 0),)),
    (2, ((2, 0),)),
    (3, ((3, 0),)),
    (4, ((6, 0),)),
    (5, ((4, 1),)),
    (6, ((5, 1),)),
)
_G_BC = (
    (0, ((0, 0), (1, 1))),
    (1, ((2, 0), (3, 1))),
    (2, ((4, 0), (5, 1))),
)
_G_HR = (
    (0, ((0, 0),)),
    (1, ((1, 0),)),
    (2, ((2, 1),)),
    (3, ((3, 1),)),
)


_EC = 8192
_NEB = (N + _EC - 1) // _EC


def _repack_body(aT, bT, p):
    p[:, :D] = aT[...].T
    p[:, D:] = bT[...].T


def _repack(aT, bT):
    return pl.pallas_call(
        _repack_body,
        grid=(_NEB,),
        in_specs=[pl.BlockSpec((D, _EC), lambda i: (0, i))] * 2,
        out_specs=pl.BlockSpec((_EC, 2 * D), lambda i: (i, 0)),
        out_shape=jax.ShapeDtypeStruct((N, 2 * D), jnp.float32),
        compiler_params=pltpu.CompilerParams(
            dimension_semantics=("arbitrary",),
        ),
    )(aT, bT)


def _make_sc_gather(gathers, n_idx, n_out):
    @functools.partial(
        pl.kernel,
        out_type=jax.ShapeDtypeStruct((n_out, BH, 2 * D), jnp.float32),
        mesh=_MESH,
        scratch_types=[
            pltpu.VMEM((n_idx, PER_W), jnp.int32),
            pltpu.VMEM((2, CH, 2 * D), jnp.float32),
            pltpu.SemaphoreType.DMA,
            pltpu.SemaphoreType.DMA,
            pltpu.SemaphoreType.DMA,
        ],
        compiler_params=pltpu.CompilerParams(use_tc_tiling_on_sc=False),
    )
    def sc_gather(tbl, idx_hbm, out, idx_v, rows, sem_i, sem_a, sem_b):
        wid = lax.axis_index("s") * NC + lax.axis_index("c")
        base = wid * PER_W
        half = base // BH
        row0 = base - half * BH
        pltpu.async_copy(idx_hbm.at[:, pl.ds(base, PER_W)], idx_v, sem_i
                         ).wait()
        for ii, outs in gathers:

            @pl.loop(0, NCH, step=2)
            def _(c, ii=ii, outs=outs):
                o0 = c * CH
                o1 = o0 + CH
                cp_a = pltpu.async_copy(
                    tbl.at[idx_v.at[ii, pl.ds(o0, CH)]], rows.at[0], sem_a)
                cp_b = pltpu.async_copy(
                    tbl.at[idx_v.at[ii, pl.ds(o1, CH)]], rows.at[1], sem_b)
                cp_a.wait()
                for g, h in outs:
                    pltpu.sync_copy(
                        rows.at[0, :, pl.ds(h * D, D)],
                        out.at[g, pl.ds(row0 + o0, CH),
                               pl.ds(half * D, D)])
                cp_b.wait()
                for g, h in outs:
                    pltpu.sync_copy(
                        rows.at[1, :, pl.ds(h * D, D)],
                        out.at[g, pl.ds(row0 + o1, CH),
                               pl.ds(half * D, D)])

    return sc_gather


_sc_et = _make_sc_gather(_G_ET, 7, 7)
_sc_bc = _make_sc_gather(_G_BC, 3, 6)
_sc_hr = _make_sc_gather(_G_HR, 4, 4)


_RH = 512


def _halfsum(v):
    return jnp.sum(v.reshape(_RH, 2, D), axis=2).T


def _tc_batch_body(m_ref, et_ref, bc_ref, hr_ref, o1, o2, o6):
    m = m_ref[0, 0]
    one = jnp.float32(1.0)
    et = et_ref[...]
    bc = bc_ref[...]
    hr = hr_ref[...]
    aBHEE, aBTEE, nABHEE, nABTEE = et[0], et[1], et[2], et[3]
    aTailE, nTailE = et[4], et[5]
    aBCHE, aBCTE, nABCHE, nABCTE = bc[0], bc[1], bc[2], bc[3]
    aHeadE, nHeadE, aRelE, nRelE = hr[0], hr[1], hr[2], hr[3]

    tmpBE2CH = (one - aBCHE) * aBHEE
    tmpBE2CT = (one - aBCTE) * aBTEE
    tE = _halfsum(jnp.abs(aHeadE + aRelE - aTailE))
    s1 = _halfsum(tmpBE2CH * tmpBE2CH + tmpBE2CT * tmpBE2CT)
    tmpNBE2CH = (one - nABCHE) * nABHEE
    tmpNBE2CT = (one - nABCTE) * nABTEE
    ntE = _halfsum(jnp.abs(nHeadE + nRelE - nTailE))
    s2 = _halfsum(tmpNBE2CH * tmpNBE2CH + tmpNBE2CT * tmpNBE2CT)
    o1[...] = s1 + tE
    o2[...] = jnp.maximum(m - (s2 + ntE), 0.0)
    o6[...] = jnp.maximum(m + tE - ntE, 0.0)


def _tc_uniq_body(ue_ref, ubc_ref, o3, o4, o5):
    one = jnp.float32(1.0)
    uniqEE = ue_ref[0]
    uniqBCHE, uniqBCTE = ubc_ref[0], ubc_ref[1]
    tmpE = _halfsum(uniqEE * uniqEE) - one
    o3[...] = tmpE * tmpE
    tmpBCH = uniqBCHE * (one - uniqBCHE)
    tmpBCT = uniqBCTE * (one - uniqBCTE)
    o4[...] = _halfsum(tmpBCH * tmpBCH + tmpBCT * tmpBCT)
    o5[...] = (jnp.maximum(one - _halfsum(jnp.abs(uniqBCHE)), 0.0)
               + jnp.maximum(one - _halfsum(jnp.abs(uniqBCTE)), 0.0))


def _blk(n):
    return pl.BlockSpec((n, _RH, 2 * D), lambda i: (0, i, 0))


_OUT_HALVES = jax.ShapeDtypeStruct((2, BH), jnp.float32)
_OUT_SPEC = pl.BlockSpec((2, _RH), lambda i: (0, i))
_TC_PARAMS = pltpu.CompilerParams(dimension_semantics=("arbitrary",))


def _tc_batch(margin2d, et, bc, hr):
    return pl.pallas_call(
        _tc_batch_body,
        grid=(BH // _RH,),
        in_specs=[pl.BlockSpec((1, 1), lambda i: (0, 0)),
                  _blk(6), _blk(4), _blk(4)],
        out_specs=[_OUT_SPEC] * 3,
        out_shape=[_OUT_HALVES] * 3,
        compiler_params=_TC_PARAMS,
    )(margin2d, et, bc, hr)


def _tc_uniq(et, bc):
    return pl.pallas_call(
        _tc_uniq_body,
        grid=(BH // _RH,),
        in_specs=[
            pl.BlockSpec((1, _RH, 2 * D), lambda i: (6, i, 0)),
            pl.BlockSpec((2, _RH, 2 * D), lambda i: (2, i, 0)),
        ],
        out_specs=[_OUT_SPEC] * 3,
        out_shape=[_OUT_HALVES] * 3,
        compiler_params=_TC_PARAMS,
    )(et, bc)


def kernel(aBHE, aBTE, aBC, aHead, aTail, aRelation, nABHE, nABTE, nABC,
           nHead, nTail, nRelation, uniqE, uniqBC, lossMargin, device,
           entityEmbed, bConceptHEmbed, bConceptTEmbed, headEmbed,
           tailEmbed, relationEmbed):
    i32 = lambda a: a.astype(jnp.int32)
    idx_et = jnp.stack([i32(aBHE), i32(aBTE), i32(nABHE), i32(nABTE),
                        i32(uniqE), i32(aTail), i32(nTail)])
    idx_bc = jnp.stack([i32(aBC), i32(nABC), i32(uniqBC)])
    idx_hr = jnp.stack([i32(aHead), i32(nHead), i32(aRelation),
                        i32(nRelation)])
    sw = lambda t: jnp.swapaxes(t, 0, 1)
    p_et = _repack(sw(entityEmbed), sw(tailEmbed))
    g_et = _sc_et(p_et, idx_et)
    p_bc = _repack(sw(bConceptHEmbed), sw(bConceptTEmbed))
    g_bc = _sc_bc(p_bc, idx_bc)
    p_hr = _repack(sw(headEmbed), sw(relationEmbed))
    g_hr = _sc_hr(p_hr, idx_hr)
    margin2d = jnp.asarray(lossMargin, jnp.float32).reshape(1, 1)
    o3, o4, o5 = _tc_uniq(g_et, g_bc)
    o1, o2, o6 = _tc_batch(margin2d, g_et, g_bc, g_hr)
    join = lambda t: jnp.concatenate([t[0], t[1]])
    return (join(o1), join(o2), join(o3), join(o4), join(o5), join(o6))


# speedup vs baseline: 1.3585x; 1.1491x over previous
import functools

import jax
import jax.numpy as jnp
from jax import lax
from jax.experimental import pallas as pl
from jax.experimental.pallas import tpu as pltpu
from jax.experimental.pallas import tpu_sc as plsc

D = 64
N = 100000
B = 16384
BH = B // 2
NC = 2
NS = 16
NW = NC * NS
PER_W = B // NW
CH = 128
NCH = PER_W // CH

_MESH = plsc.VectorSubcoreMesh(
    core_axis_name="c", subcore_axis_name="s", num_cores=NC, num_subcores=NS
)

_G_ET = (
    (0, ((0, 0),)),
    (1, ((1, 0),)),
    (2, ((2, 0),)),
    (3, ((3, 0),)),
    (4, ((6, 0),)),
    (5, ((4, 1),)),
    (6, ((5, 1),)),
)
_G_BC = (
    (0, ((0, 0), (1, 1))),
    (1, ((2, 0), (3, 1))),
    (2, ((4, 0), (5, 1))),
)
_G_HR = (
    (0, ((0, 0),)),
    (1, ((1, 0),)),
    (2, ((2, 1),)),
    (3, ((3, 1),)),
)


_EC = 8192
_NEB = (N + _EC - 1) // _EC


def _repack_body(aT, bT, p):
    p[:, :D] = aT[...].T
    p[:, D:] = bT[...].T


def _repack(aT, bT):
    return pl.pallas_call(
        _repack_body,
        grid=(_NEB,),
        in_specs=[pl.BlockSpec((D, _EC), lambda i: (0, i))] * 2,
        out_specs=pl.BlockSpec((_EC, 2 * D), lambda i: (i, 0)),
        out_shape=jax.ShapeDtypeStruct((N, 2 * D), jnp.float32),
        compiler_params=pltpu.CompilerParams(
            dimension_semantics=("arbitrary",),
        ),
    )(aT, bT)


def _make_sc_gather(gathers, n_idx, n_out):
    @functools.partial(
        pl.kernel,
        out_type=jax.ShapeDtypeStruct((n_out, BH, 2 * D), jnp.float32),
        mesh=_MESH,
        scratch_types=[
            pltpu.VMEM((n_idx, PER_W), jnp.int32),
            pltpu.VMEM((2, CH, 2 * D), jnp.float32),
            pltpu.SemaphoreType.DMA,
            pltpu.SemaphoreType.DMA,
            pltpu.SemaphoreType.DMA,
        ],
        compiler_params=pltpu.CompilerParams(use_tc_tiling_on_sc=False),
    )
    def sc_gather(tbl, idx_hbm, out, idx_v, rows, sem_i, sem_a, sem_b):
        wid = lax.axis_index("s") * NC + lax.axis_index("c")
        base = wid * PER_W
        half = base // BH
        row0 = base - half * BH
        pltpu.async_copy(idx_hbm.at[:, pl.ds(base, PER_W)], idx_v, sem_i
                         ).wait()
        for ii, outs in gathers:

            @pl.loop(0, NCH, step=2)
            def _(c, ii=ii, outs=outs):
                o0 = c * CH
                o1 = o0 + CH
                cp_a = pltpu.async_copy(
                    tbl.at[idx_v.at[ii, pl.ds(o0, CH)]], rows.at[0], sem_a)
                cp_b = pltpu.async_copy(
                    tbl.at[idx_v.at[ii, pl.ds(o1, CH)]], rows.at[1], sem_b)
                cp_a.wait()
                for g, h in outs:
                    pltpu.sync_copy(
                        rows.at[0, :, pl.ds(h * D, D)],
                        out.at[g, pl.ds(row0 + o0, CH),
                               pl.ds(half * D, D)])
                cp_b.wait()
                for g, h in outs:
                    pltpu.sync_copy(
                        rows.at[1, :, pl.ds(h * D, D)],
                        out.at[g, pl.ds(row0 + o1, CH),
                               pl.ds(half * D, D)])

    return sc_gather


_sc_et = _make_sc_gather(_G_ET, 7, 7)
_sc_bc = _make_sc_gather(_G_BC, 3, 6)
_sc_hr = _make_sc_gather(_G_HR, 4, 4)


_RH = 512


def _half_mask():
    row = lax.broadcasted_iota(jnp.int32, (2, 2 * D), 0)
    lane = lax.broadcasted_iota(jnp.int32, (2, 2 * D), 1)
    return jnp.where((lane < D) == (row == 0), jnp.float32(1), jnp.float32(0))


def _halfsum(v, mask):
    return lax.dot_general(
        mask, v, (((1,), (1,)), ((), ())),
        preferred_element_type=jnp.float32,
        precision=lax.Precision.HIGHEST)


def _tc_batch_body(m_ref, et_ref, bc_ref, hr_ref, o1, o2, o6):
    m = m_ref[0, 0]
    one = jnp.float32(1.0)
    et = et_ref[...]
    bc = bc_ref[...]
    hr = hr_ref[...]
    aBHEE, aBTEE, nABHEE, nABTEE = et[0], et[1], et[2], et[3]
    aTailE, nTailE = et[4], et[5]
    aBCHE, aBCTE, nABCHE, nABCTE = bc[0], bc[1], bc[2], bc[3]
    aHeadE, nHeadE, aRelE, nRelE = hr[0], hr[1], hr[2], hr[3]

    msk = _half_mask()
    tmpBE2CH = (one - aBCHE) * aBHEE
    tmpBE2CT = (one - aBCTE) * aBTEE
    tE = _halfsum(jnp.abs(aHeadE + aRelE - aTailE), msk)
    s1 = _halfsum(tmpBE2CH * tmpBE2CH + tmpBE2CT * tmpBE2CT, msk)
    tmpNBE2CH = (one - nABCHE) * nABHEE
    tmpNBE2CT = (one - nABCTE) * nABTEE
    ntE = _halfsum(jnp.abs(nHeadE + nRelE - nTailE), msk)
    s2 = _halfsum(tmpNBE2CH * tmpNBE2CH + tmpNBE2CT * tmpNBE2CT, msk)
    o1[...] = s1 + tE
    o2[...] = jnp.maximum(m - (s2 + ntE), 0.0)
    o6[...] = jnp.maximum(m + tE - ntE, 0.0)


def _tc_uniq_body(ue_ref, ubc_ref, o3, o4, o5):
    one = jnp.float32(1.0)
    uniqEE = ue_ref[0]
    uniqBCHE, uniqBCTE = ubc_ref[0], ubc_ref[1]
    msk = _half_mask()
    tmpE = _halfsum(uniqEE * uniqEE, msk) - one
    o3[...] = tmpE * tmpE
    tmpBCH = uniqBCHE * (one - uniqBCHE)
    tmpBCT = uniqBCTE * (one - uniqBCTE)
    o4[...] = _halfsum(tmpBCH * tmpBCH + tmpBCT * tmpBCT, msk)
    o5[...] = (jnp.maximum(one - _halfsum(jnp.abs(uniqBCHE), msk), 0.0)
               + jnp.maximum(one - _halfsum(jnp.abs(uniqBCTE), msk), 0.0))


def _blk(n):
    return pl.BlockSpec((n, _RH, 2 * D), lambda i: (0, i, 0))


_OUT_HALVES = jax.ShapeDtypeStruct((2, BH), jnp.float32)
_OUT_SPEC = pl.BlockSpec((2, _RH), lambda i: (0, i))
_TC_PARAMS = pltpu.CompilerParams(dimension_semantics=("arbitrary",))


def _tc_batch(margin2d, et, bc, hr):
    return pl.pallas_call(
        _tc_batch_body,
        grid=(BH // _RH,),
        in_specs=[pl.BlockSpec((1, 1), lambda i: (0, 0)),
                  _blk(6), _blk(4), _blk(4)],
        out_specs=[_OUT_SPEC] * 3,
        out_shape=[_OUT_HALVES] * 3,
        compiler_params=_TC_PARAMS,
    )(margin2d, et, bc, hr)


def _tc_uniq(et, bc):
    return pl.pallas_call(
        _tc_uniq_body,
        grid=(BH // _RH,),
        in_specs=[
            pl.BlockSpec((1, _RH, 2 * D), lambda i: (6, i, 0)),
            pl.BlockSpec((2, _RH, 2 * D), lambda i: (2, i, 0)),
        ],
        out_specs=[_OUT_SPEC] * 3,
        out_shape=[_OUT_HALVES] * 3,
        compiler_params=_TC_PARAMS,
    )(et, bc)


def kernel(aBHE, aBTE, aBC, aHead, aTail, aRelation, nABHE, nABTE, nABC,
           nHead, nTail, nRelation, uniqE, uniqBC, lossMargin, device,
           entityEmbed, bConceptHEmbed, bConceptTEmbed, headEmbed,
           tailEmbed, relationEmbed):
    i32 = lambda a: a.astype(jnp.int32)
    idx_et = jnp.stack([i32(aBHE), i32(aBTE), i32(nABHE), i32(nABTE),
                        i32(uniqE), i32(aTail), i32(nTail)])
    idx_bc = jnp.stack([i32(aBC), i32(nABC), i32(uniqBC)])
    idx_hr = jnp.stack([i32(aHead), i32(nHead), i32(aRelation),
                        i32(nRelation)])
    sw = lambda t: jnp.swapaxes(t, 0, 1)
    p_et = _repack(sw(entityEmbed), sw(tailEmbed))
    g_et = _sc_et(p_et, idx_et)
    p_bc = _repack(sw(bConceptHEmbed), sw(bConceptTEmbed))
    g_bc = _sc_bc(p_bc, idx_bc)
    p_hr = _repack(sw(headEmbed), sw(relationEmbed))
    g_hr = _sc_hr(p_hr, idx_hr)
    margin2d = jnp.asarray(lossMargin, jnp.float32).reshape(1, 1)
    o3, o4, o5 = _tc_uniq(g_et, g_bc)
    o1, o2, o6 = _tc_batch(margin2d, g_et, g_bc, g_hr)
    join = lambda t: jnp.concatenate([t[0], t[1]])
    return (join(o1), join(o2), join(o3), join(o4), join(o5), join(o6))
